# baseline (speedup 1.0000x reference)
import functools

import jax
import jax.numpy as jnp
from jax import lax
from jax.experimental import pallas as pl
from jax.experimental.pallas import tpu as pltpu
from jax.experimental.pallas import tpu_sc as plsc

N_USERS = 50000
N_NODES = 100000
HALF = 16
N_LAYERS = 3
N_EDGES = 1600000
BATCH = 4096
N_PAD = 100096

EPG = 384
IDXW = 128
SUBG = EPG // IDXW
GROUPS = 264
EPT = GROUPS * EPG
ROWS_PT = EPT // IDXW
N_E_PAD = EPT * 16
N_E_ALLOC = N_E_PAD + 2 * EPG
NODES_PT = N_PAD // 16
PAIRS_PT = BATCH // 16


def kernel(user_table, item_table, edge_values, user_idx, item_idx, edge_index):
  f32 = jnp.float32
  i32 = jnp.int32

  all_emb = jnp.concatenate([user_table, item_table], axis=0)
  x0h = all_emb.reshape(N_NODES, 2, HALF).transpose(1, 0, 2)

  src = edge_index[0].astype(i32)
  dst = edge_index[1].astype(i32)
  val = edge_values.astype(f32)
  pad = N_E_ALLOC - N_EDGES
  srcf = jnp.concatenate([src, jnp.zeros((pad,), i32)])
  valf = jnp.concatenate([val, jnp.zeros((pad,), f32)])
  dstr = jnp.concatenate([dst, jnp.zeros((pad,), i32)]).reshape(-1, IDXW)

  uidx = user_idx.astype(i32)
  iidx = item_idx.astype(i32) + N_USERS

  mesh = plsc.VectorSubcoreMesh(core_axis_name="c", subcore_axis_name="s")

  @functools.partial(
      pl.kernel,
      out_type=[
          jax.ShapeDtypeStruct((2, BATCH, HALF), f32),
          jax.ShapeDtypeStruct((2, BATCH, HALF), f32),
          jax.ShapeDtypeStruct((2, N_LAYERS, N_PAD, HALF), f32),
      ],
      mesh=mesh,
      compiler_params=pltpu.CompilerParams(use_tc_tiling_on_sc=False),
      scratch_types=[
          pltpu.VMEM((3, EPG), i32),
          pltpu.VMEM((3, SUBG, IDXW), i32),
          pltpu.VMEM((3, EPG), f32),
          pltpu.VMEM((3, EPG, HALF), f32),
          pltpu.VMEM((IDXW,), i32),
          pltpu.VMEM((IDXW, HALF), f32),
          pltpu.VMEM((IDXW, HALF), f32),
          pltpu.VMEM_SHARED((N_PAD, HALF), f32),
          pltpu.SemaphoreType.DMA((3,)),
          pltpu.SemaphoreType.DMA((3,)),
          pltpu.SemaphoreType.DMA((3,)),
          pltpu.SemaphoreType.DMA((3,)),
          pltpu.SemaphoreType.DMA,
      ],
  )
  def lightgcn(x0_hbm, srcf_hbm, dstr_hbm, valf_hbm, uidx_hbm, iidx_hbm,
               ug_hbm, ig_hbm, xs_hbm,
               srcv, dstv, valv, rows, fidx, fb, fgb, acc,
               gsem, ssem, isem, dsem, asem):
    c = lax.axis_index("c")
    s = lax.axis_index("s")
    node_base = s * NODES_PT

    def srcval_cps(sl, g):
      ebase = s * EPT + g * EPG
      sm = isem.at[sl]
      return [
          pltpu.make_async_copy(srcf_hbm.at[pl.ds(ebase, EPG)],
                                srcv.at[sl], sm),
          pltpu.make_async_copy(valf_hbm.at[pl.ds(ebase, EPG)],
                                valv.at[sl], sm),
      ]

    def dstidx_cps(sl, g):
      rbase = s * ROWS_PT + g * SUBG
      return [pltpu.make_async_copy(dstr_hbm.at[pl.ds(rbase, SUBG)],
                                    dstv.at[sl], dsem.at[sl])]

    def gather_cps(b, sl, x_src):
      return [
          pltpu.make_async_copy(
              x_src.at[srcv.at[sl, pl.ds(jj * IDXW, IDXW)]],
              rows.at[b, pl.ds(jj * IDXW, IDXW)], gsem.at[b])
          for jj in range(SUBG)
      ]

    def scatter_cps(b, sl):
      return [
          pltpu.make_async_copy(
              rows.at[b, pl.ds(jj * IDXW, IDXW)],
              acc.at[dstv.at[sl, jj]], ssem.at[b])
          for jj in range(SUBG)
      ]

    def issue(cps, add=False):
      for cp in cps:
        cp.start(add=add)

    def drain(cps):
      for cp in cps:
        cp.wait()

    def multiply(b, sl):
      @pl.loop(0, EPG // 16)
      def _(t):
        vv = valv[sl, pl.ds(t * 16, 16)]
        e0 = t * 16
        for i in range(16):
          rows[b, e0 + i, :] = rows[b, e0 + i, :] * vv[i]

    def zero_rows0():
      @pl.loop(0, EPG // SUBG)
      def _(t):
        for q in range(SUBG):
          rows[0, t * SUBG + q, :] = jnp.zeros((HALF,), f32)

    def zero_acc_cps():
      nfull = NODES_PT // EPG
      rem = NODES_PT - nfull * EPG
      cps = [pltpu.make_async_copy(
          rows.at[0], acc.at[pl.ds(node_base + q * EPG, EPG)], asem)
          for q in range(nfull)]
      cps.append(pltpu.make_async_copy(
          rows.at[0, pl.ds(0, rem)],
          acc.at[pl.ds(node_base + nfull * EPG, rem)], asem))
      return cps

    def zero_acc():
      zcps = zero_acc_cps()
      issue(zcps)
      drain(zcps)

    def edge_pipeline(x_src):
      issue(srcval_cps(0, 0))
      issue(srcval_cps(1, 1))
      issue(dstidx_cps(0, 0))
      drain(srcval_cps(0, 0))
      issue(gather_cps(0, 0, x_src))

      def body(g, bi, first):
        b = bi % 3
        b1 = (bi + 1) % 3
        b2 = (bi + 2) % 3
        drain(gather_cps(b, b, x_src))
        if first != 0:
          drain(scatter_cps(b1, b1))
        issue(dstidx_cps(b1, g + 1))
        drain(srcval_cps(b1, g + 1))
        issue(gather_cps(b1, b1, x_src))
        issue(srcval_cps(b2, g + 2))
        drain(dstidx_cps(b, g))
        multiply(b, b)
        issue(scatter_cps(b, b), add=True)

      body(0, 0, 0)
      body(1, 1, 0)
      body(2, 2, 1)

      @pl.loop(3, GROUPS, step=3)
      def _(g3):
        for ii in range(3):
          body(g3 + ii, ii, 1)

      drain(gather_cps(0, 0, x_src))
      drain(srcval_cps(1, GROUPS + 1))
      drain(dstidx_cps(0, GROUPS))
      drain(scatter_cps(1, 1))
      drain(scatter_cps(2, 2))

    zero_rows0()
    zero_acc()
    plsc.subcore_barrier()

    def finish_layer(dst_ref):
      plsc.subcore_barrier()
      pltpu.sync_copy(acc.at[pl.ds(node_base, NODES_PT)], dst_ref)
      zero_rows0()
      zero_acc()
      plsc.subcore_barrier()

    edge_pipeline(x0_hbm.at[c])
    finish_layer(xs_hbm.at[c, 0, pl.ds(node_base, NODES_PT)])

    @pl.loop(1, N_LAYERS)
    def _(k):
      edge_pipeline(xs_hbm.at[c, k - 1])
      finish_layer(xs_hbm.at[c, k, pl.ds(node_base, NODES_PT)])

    def gather_mean(nidx_hbm, out_hbm):
      for chunk in range(PAIRS_PT // IDXW):
        pbase = s * PAIRS_PT + chunk * IDXW
        pltpu.sync_copy(nidx_hbm.at[pl.ds(pbase, IDXW)], fidx)
        pltpu.sync_copy(x0_hbm.at[c].at[fidx], fb)
        for k in range(N_LAYERS):
          pltpu.sync_copy(xs_hbm.at[c, k].at[fidx], fgb)

          @pl.loop(0, IDXW, unroll=8)
          def _(p):
            fb[p, :] = fb[p, :] + fgb[p, :]

        pltpu.sync_copy(fb, out_hbm.at[c, pl.ds(pbase, IDXW)])

    gather_mean(uidx_hbm, ug_hbm)
    gather_mean(iidx_hbm, ig_hbm)

  ug, ig, _ = lightgcn(x0h, srcf, dstr, valf, uidx, iidx)

  def dot_body(u_ref, i_ref, o_ref):
    u = u_ref[...]
    v = i_ref[...]
    o_ref[...] = (u[0] * v[0] + u[1] * v[1]).sum(axis=-1) * (1.0 / 16.0)

  scores = pl.pallas_call(
      dot_body,
      out_shape=jax.ShapeDtypeStruct((BATCH,), f32),
  )(ug, ig)
  return scores
